# baseline (speedup 1.0000x reference)
import functools

import jax
import jax.numpy as jnp
from jax import lax
from jax.experimental import pallas as pl
from jax.experimental.pallas import tpu as pltpu
from jax.experimental.pallas import tpu_sc as plsc

CHANNEL = 2048
NUM_FREQ = 4096
LEN_SEQ = 4096
N = 64
CHUNK = CHANNEL // N


def _topk_body(w_ref, idx_ref, val_ref):
    w = w_ref[...]
    m = jnp.max(w)
    e = jnp.exp(w - m)
    p = e / jnp.sum(e)
    row = lax.broadcasted_iota(jnp.int32, (32, 128), 0)
    col = lax.broadcasted_iota(jnp.int32, (32, 128), 1)
    flat = row * 128 + col

    def body(k, p):
        v = jnp.max(p)
        idx = jnp.min(jnp.where(p == v, flat, jnp.int32(1 << 30)))
        idx_ref[k // 2, k % 2] = idx
        idx_ref[k // 2, 2 + k % 2] = idx
        idx_ref[k // 2, 4 + k % 2] = idx
        idx_ref[k // 2, 6 + k % 2] = idx
        val_ref[k] = v
        return jnp.where(flat == idx, jnp.float32(-1.0), p)

    lax.fori_loop(0, N, body, p)


_topk = pl.pallas_call(
    _topk_body,
    out_shape=[
        jax.ShapeDtypeStruct((N // 2, 8), jnp.int32),
        jax.ShapeDtypeStruct((N,), jnp.float32),
    ],
    out_specs=[
        pl.BlockSpec(memory_space=pltpu.SMEM),
        pl.BlockSpec(memory_space=pltpu.SMEM),
    ],
)

_mesh = plsc.VectorSubcoreMesh(core_axis_name="c", subcore_axis_name="s")


@functools.partial(
    pl.kernel,
    mesh=_mesh,
    out_type=jax.ShapeDtypeStruct((CHANNEL, LEN_SEQ), jnp.float32),
    scratch_types=[
        pltpu.VMEM((8,), jnp.int32),
        pltpu.VMEM((N, 16), jnp.float32),
        pltpu.VMEM((2, LEN_SEQ), jnp.float32),
        pltpu.SemaphoreType.DMA,
        pltpu.SemaphoreType.DMA,
    ],
)
def _bcast(filters_hbm, idx_hbm, vals_hbm, out_hbm, idx_v, vals_v, rows,
           gsem, ssem):
    wid = lax.axis_index("s") * 2 + lax.axis_index("c")
    pltpu.sync_copy(idx_hbm.at[wid], idx_v)
    pltpu.sync_copy(vals_hbm, vals_v)
    c0 = wid * 2
    c1 = wid * 2 + 1
    pltpu.async_copy(filters_hbm.at[idx_v.at[pl.ds(0, 2)]], rows, gsem).wait()

    def scale(r, w):
        view = rows.at[r]

        def body(j, _):
            s = view[pl.ds(j * 16, 16)]
            view[pl.ds(j * 16, 16)] = s * w
            return 0

        lax.fori_loop(0, LEN_SEQ // 16, body, 0)

    scale(0, vals_v[c0])
    cps = [
        pltpu.async_copy(rows.at[pl.ds(0, 1)],
                         out_hbm.at[pl.ds(c0 * CHUNK + j, 1)], ssem)
        for j in range(CHUNK)
    ]
    scale(1, vals_v[c1])
    cps += [
        pltpu.async_copy(rows.at[pl.ds(1, 1)],
                         out_hbm.at[pl.ds(c1 * CHUNK + j, 1)], ssem)
        for j in range(CHUNK)
    ]
    for cp in cps:
        cp.wait()


def kernel(filters, weights):
    idx, vals = _topk(weights.reshape(32, 128))
    vals_rep = jnp.broadcast_to(vals[:, None], (N, 16))
    return _bcast(filters, idx, vals_rep)


# speedup vs baseline: 1.5932x; 1.5932x over previous
import functools

import jax
import jax.numpy as jnp
from jax import lax
from jax.experimental import pallas as pl
from jax.experimental.pallas import tpu as pltpu
from jax.experimental.pallas import tpu_sc as plsc

CHANNEL = 2048
NUM_FREQ = 4096
LEN_SEQ = 4096
N = 64
CHUNK = CHANNEL // N


def _excl_prefix(m):
    x = m
    for s in (1, 2, 4, 8, 16, 32, 64):
        x = x + jnp.concatenate(
            [jnp.zeros((32, s), jnp.int32), x[:, :128 - s]], axis=1)
    row_tot = x[:, 127:128]
    y = row_tot
    for s in (1, 2, 4, 8, 16):
        y = y + jnp.concatenate(
            [jnp.zeros((s, 1), jnp.int32), y[:32 - s, :]], axis=0)
    return x - m + (y - row_tot)


def _topk_body(w_ref, idx_ref, val_ref):
    w = w_ref[...]
    mx = jnp.max(w)
    e = jnp.exp(w - mx)
    p = e / jnp.sum(e)
    row = lax.broadcasted_iota(jnp.int32, (32, 128), 0)
    col = lax.broadcasted_iota(jnp.int32, (32, 128), 1)
    flat = row * 128 + col
    bits = lax.bitcast_convert_type(p, jnp.int32)

    def cnt_ge(t):
        return jnp.sum((bits > t).astype(jnp.int32), keepdims=True) >= N

    def bis(_, lohi):
        lo, hi = lohi
        e = jnp.maximum(hi - lo >> 3, 1)
        ms = [lo + e * k for k in range(1, 8)]
        bs = [cnt_ge(m) for m in ms]
        lo2 = lo
        hi2 = ms[0]
        for k in range(7):
            up = ms[k + 1] if k < 6 else hi
            lo2 = jnp.where(bs[k], ms[k] + 1, lo2)
            hi2 = jnp.where(bs[k], up, hi2)
        return lo2, hi2

    lo0 = lax.bitcast_convert_type(
        jnp.min(jnp.max(p, axis=0, keepdims=True), axis=1, keepdims=True),
        jnp.int32)
    hi0 = jnp.full((1, 1), 0x3C800000, jnp.int32)
    T, _ = lax.fori_loop(0, 12, bis, (lo0, hi0), unroll=True)

    gt = bits > T
    eq = bits == T
    cnt_gt = jnp.sum(gt.astype(jnp.int32), keepdims=True)
    need = N - cnt_gt
    tie_rank = _excl_prefix(eq.astype(jnp.int32))
    winner = gt | (eq & (tie_rank < need))
    pos = _excl_prefix(winner.astype(jnp.int32))
    posm = jnp.where(winner, pos, jnp.int32(127))
    pv = jnp.where(winner, p, jnp.float32(0.0))
    fm = jnp.where(winner, flat, jnp.int32(0))

    kcol = lax.broadcasted_iota(jnp.int32, (N, 128), 0)
    accv = jnp.zeros((N, 128), jnp.float32)
    acci = jnp.zeros((N, 128), jnp.int32)
    for r in range(32):
        hit = posm[r:r + 1, :] == kcol
        accv = accv + jnp.where(hit, pv[r:r + 1, :], 0.0)
        acci = acci + jnp.where(hit, fm[r:r + 1, :], 0)
    compv = jnp.sum(accv, axis=1, keepdims=True)
    compi = jnp.sum(acci, axis=1, keepdims=True)

    r64 = lax.broadcasted_iota(jnp.int32, (N, N), 0)
    c64 = lax.broadcasted_iota(jnp.int32, (N, N), 1)
    diag = r64 == c64

    def col2row_f(xc):
        return jnp.sum(jnp.where(diag, xc, 0.0), axis=0, keepdims=True)

    def col2row_i(xc):
        return jnp.sum(jnp.where(diag, xc, 0), axis=0, keepdims=True)

    vrow = col2row_f(compv)
    irow = col2row_i(compi)

    beats = (vrow > compv) | ((vrow == compv) & (irow < compi))
    rank = jnp.sum(beats.astype(jnp.int32), axis=1, keepdims=True)
    rank_row = col2row_i(rank)

    selv = jnp.where(rank_row == r64, jnp.broadcast_to(vrow, (N, N)), 0.0)
    val_sorted = jnp.sum(selv, axis=1, keepdims=True)
    val_ref[...] = jnp.broadcast_to(val_sorted, (N, 16))

    k3 = lax.broadcasted_iota(jnp.int32, (N, 8, 1), 0)
    sel3 = jnp.where(rank_row.reshape(1, 1, N) == k3,
                     irow.reshape(1, 1, N), 0)
    idx_ref[...] = jnp.sum(sel3, axis=2)


_topk = pl.pallas_call(
    _topk_body,
    out_shape=[
        jax.ShapeDtypeStruct((N, 8), jnp.int32),
        jax.ShapeDtypeStruct((N, 16), jnp.float32),
    ],
)

_mesh = plsc.VectorSubcoreMesh(core_axis_name="c", subcore_axis_name="s")


@functools.partial(
    pl.kernel,
    mesh=_mesh,
    out_type=jax.ShapeDtypeStruct((CHANNEL, LEN_SEQ), jnp.float32),
    scratch_types=[
        pltpu.VMEM((8,), jnp.int32),
        pltpu.VMEM((16,), jnp.float32),
        pltpu.VMEM((1, LEN_SEQ), jnp.float32),
        pltpu.VMEM((1, LEN_SEQ), jnp.float32),
        pltpu.SemaphoreType.DMA,
        pltpu.SemaphoreType.DMA,
        pltpu.SemaphoreType.DMA,
    ],
)
def _bcast(filters_hbm, idx_hbm, vals_hbm, out_hbm, idx_v, vals_v, row_a,
           row_b, gsem, sem_a, sem_b):
    wid = lax.axis_index("s") * 2 + lax.axis_index("c")

    def do_chunk(c, rows, ssem):
        pltpu.sync_copy(idx_hbm.at[c], idx_v)
        cp = pltpu.async_copy(filters_hbm.at[idx_v.at[pl.ds(0, 1)]], rows,
                              gsem)
        pltpu.sync_copy(vals_hbm.at[c], vals_v)
        cp.wait()
        w = vals_v[...]
        view = rows.at[0]

        def body(j, _):
            s = view[pl.ds(j * 16, 16)]
            view[pl.ds(j * 16, 16)] = s * w
            return 0

        lax.fori_loop(0, LEN_SEQ // 16, body, 0)
        return [
            pltpu.async_copy(rows, out_hbm.at[pl.ds(c * CHUNK + j, 1)], ssem)
            for j in range(CHUNK)
        ]

    cps_a = do_chunk(wid * 2, row_a, sem_a)
    cps_b = do_chunk(wid * 2 + 1, row_b, sem_b)
    for cp in cps_a:
        cp.wait()
    for cp in cps_b:
        cp.wait()


def kernel(filters, weights):
    idx, vals = _topk(weights.reshape(32, 128))
    return _bcast(filters, idx, vals)
